# baseline (speedup 1.0000x reference)
import functools

import jax
import jax.numpy as jnp
from jax import lax
from jax.experimental import pallas as pl
from jax.experimental.pallas import tpu as pltpu
from jax.experimental.pallas import tpu_sc as plsc

_D = 64
_B = 16384
_L = 16
_NC = 2
_NS = 16
_NW = _NC * _NS
_BPW = _B // _NW
_CHUNK = 128
_NCHUNK = _BPW // _CHUNK

_mesh = plsc.VectorSubcoreMesh(core_axis_name="c", subcore_axis_name="s")


@functools.partial(
    pl.kernel,
    mesh=_mesh,
    out_type=jax.ShapeDtypeStruct((_B,), jnp.float32),
    compiler_params=pltpu.CompilerParams(
        needs_layout_passes=False, use_tc_tiling_on_sc=False
    ),
    scratch_types=[
        pltpu.VMEM((_NCHUNK, _CHUNK), jnp.int32),
        pltpu.VMEM((_NCHUNK, _CHUNK), jnp.int32),
        pltpu.VMEM((_NCHUNK, _CHUNK), jnp.int32),
        pltpu.VMEM((_NCHUNK, _CHUNK, _D), jnp.float32),
        pltpu.VMEM((_NCHUNK, _CHUNK, _D), jnp.float32),
        pltpu.VMEM((_NCHUNK, _CHUNK, _D), jnp.float32),
        pltpu.VMEM((_BPW,), jnp.float32),
        pltpu.SemaphoreType.DMA,
        pltpu.SemaphoreType.DMA,
        pltpu.SemaphoreType.DMA,
        pltpu.SemaphoreType.DMA,
    ],
)
def _bt_norm_kernel(i_hbm, j_hbm, k_hbm, u_hbm, v_hbm, out_hbm,
                    ii_v, jj_v, kk_v, u_v, vj_v, vk_v, o_v,
                    sem0, sem1, sem2, sem3):
    sems = (sem0, sem1, sem2, sem3)
    wid = lax.axis_index("s") * _NC + lax.axis_index("c")
    base = wid * _BPW

    for c in range(_NCHUNK):
        off = base + c * _CHUNK
        pltpu.sync_copy(i_hbm.at[pl.ds(off, _CHUNK)], ii_v.at[c])
        pltpu.sync_copy(j_hbm.at[pl.ds(off, _CHUNK)], jj_v.at[c])
        pltpu.sync_copy(k_hbm.at[pl.ds(off, _CHUNK)], kk_v.at[c])

    copies = []
    for c in range(_NCHUNK):
        copies.append((
            pltpu.async_copy(u_hbm.at[ii_v.at[c]], u_v.at[c], sems[c]),
            pltpu.async_copy(v_hbm.at[jj_v.at[c]], vj_v.at[c], sems[c]),
            pltpu.async_copy(v_hbm.at[kk_v.at[c]], vk_v.at[c], sems[c]),
        ))

    lane = lax.iota(jnp.int32, _L)
    zero = jnp.zeros((_L,), jnp.float32)

    for c in range(_NCHUNK):
        for cp in copies[c]:
            cp.wait()

        def group_body(g, _, c=c):
            row0 = g * _L
            rows = row0 + lane

            def col_body(col, acc):
                cvec = jnp.full((_L,), col, jnp.int32)
                u16 = plsc.load_gather(u_v.at[c], [rows, cvec])
                vj16 = plsc.load_gather(vj_v.at[c], [rows, cvec])
                vk16 = plsc.load_gather(vk_v.at[c], [rows, cvec])
                dj = u16 - vj16
                dk = u16 - vk16
                return acc + (dk * dk - dj * dj)

            t = lax.fori_loop(0, _D, col_body, zero, unroll=4)
            sig = 1.0 / (1.0 + jnp.exp(-t))
            o_v[pl.ds(c * _CHUNK + row0, _L)] = sig
            return 0

        lax.fori_loop(0, _CHUNK // _L, group_body, 0)

    pltpu.sync_copy(o_v, out_hbm.at[pl.ds(base, _BPW)])


def kernel(i, j, k, u_weight, v_weight):
    return _bt_norm_kernel(
        i.astype(jnp.int32),
        j.astype(jnp.int32),
        k.astype(jnp.int32),
        u_weight,
        v_weight,
    )


# speedup vs baseline: 1.1399x; 1.1399x over previous
import functools

import jax
import jax.numpy as jnp
from jax import lax
from jax.experimental import pallas as pl
from jax.experimental.pallas import tpu as pltpu
from jax.experimental.pallas import tpu_sc as plsc

_D = 64
_B = 16384
_L = 16
_NC = 2
_NS = 16
_NW = _NC * _NS
_BPW = _B // _NW
_CHUNK = 128
_NCHUNK = _BPW // _CHUNK

_mesh = plsc.VectorSubcoreMesh(core_axis_name="c", subcore_axis_name="s")


@functools.partial(
    pl.kernel,
    mesh=_mesh,
    out_type=jax.ShapeDtypeStruct((_B,), jnp.float32),
    compiler_params=pltpu.CompilerParams(
        needs_layout_passes=False, use_tc_tiling_on_sc=False
    ),
    scratch_types=[
        pltpu.VMEM((_NCHUNK, _CHUNK), jnp.int32),
        pltpu.VMEM((_NCHUNK, _CHUNK), jnp.int32),
        pltpu.VMEM((_NCHUNK, _CHUNK), jnp.int32),
        pltpu.VMEM((_NCHUNK, _CHUNK, _D), jnp.float32),
        pltpu.VMEM((_NCHUNK, _CHUNK, _D), jnp.float32),
        pltpu.VMEM((_NCHUNK, _CHUNK, _D), jnp.float32),
        pltpu.VMEM((_BPW,), jnp.float32),
        pltpu.SemaphoreType.DMA,
        pltpu.SemaphoreType.DMA,
        pltpu.SemaphoreType.DMA,
        pltpu.SemaphoreType.DMA,
    ],
)
def _bt_norm_kernel(i_hbm, j_hbm, k_hbm, u_hbm, v_hbm, out_hbm,
                    ii_v, jj_v, kk_v, u_v, vj_v, vk_v, o_v,
                    sem0, sem1, sem2, sem3):
    sems = (sem0, sem1, sem2, sem3)
    wid = lax.axis_index("s") * _NC + lax.axis_index("c")
    base = wid * _BPW

    for c in range(_NCHUNK):
        off = base + c * _CHUNK
        pltpu.sync_copy(i_hbm.at[pl.ds(off, _CHUNK)], ii_v.at[c])
        pltpu.sync_copy(j_hbm.at[pl.ds(off, _CHUNK)], jj_v.at[c])
        pltpu.sync_copy(k_hbm.at[pl.ds(off, _CHUNK)], kk_v.at[c])

    copies = []
    for c in range(_NCHUNK):
        copies.append((
            pltpu.async_copy(u_hbm.at[ii_v.at[c]], u_v.at[c], sems[c]),
            pltpu.async_copy(v_hbm.at[jj_v.at[c]], vj_v.at[c], sems[c]),
            pltpu.async_copy(v_hbm.at[kk_v.at[c]], vk_v.at[c], sems[c]),
        ))

    lane = lax.iota(jnp.int32, _L)
    hi_mask = lane == (_L - 1)

    for c in range(_NCHUNK):
        for cp in copies[c]:
            cp.wait()

        def row_body(r, _, c=c):
            acc = jnp.zeros((_L,), jnp.float32)
            for q in range(_D // _L):
                sl = pl.ds(q * _L, _L)
                u16 = u_v[c, r, sl]
                dj = u16 - vj_v[c, r, sl]
                dk = u16 - vk_v[c, r, sl]
                acc = acc + (dk * dk - dj * dj)
            cum = plsc.cumsum(acc)
            sig = 1.0 / (1.0 + jnp.exp(-cum))
            pos = jnp.full((_L,), c * _CHUNK + r, jnp.int32)
            plsc.store_scatter(o_v, [pos], sig, mask=hi_mask)
            return 0

        lax.fori_loop(0, _CHUNK, row_body, 0, unroll=4)

    pltpu.sync_copy(o_v, out_hbm.at[pl.ds(base, _BPW)])


def kernel(i, j, k, u_weight, v_weight):
    return _bt_norm_kernel(
        i.astype(jnp.int32),
        j.astype(jnp.int32),
        k.astype(jnp.int32),
        u_weight,
        v_weight,
    )


# speedup vs baseline: 1.1794x; 1.0347x over previous
import functools

import jax
import jax.numpy as jnp
from jax import lax
from jax.experimental import pallas as pl
from jax.experimental.pallas import tpu as pltpu
from jax.experimental.pallas import tpu_sc as plsc

_D = 64
_B = 16384
_L = 16
_NC = 2
_NS = 16
_NW = _NC * _NS
_BPW = _B // _NW
_CHUNK = 256
_NCHUNK = _BPW // _CHUNK

_mesh = plsc.VectorSubcoreMesh(core_axis_name="c", subcore_axis_name="s")


@functools.partial(
    pl.kernel,
    mesh=_mesh,
    out_type=jax.ShapeDtypeStruct((_B,), jnp.float32),
    compiler_params=pltpu.CompilerParams(
        needs_layout_passes=False, use_tc_tiling_on_sc=False
    ),
    scratch_types=[
        pltpu.VMEM((_NCHUNK, _CHUNK), jnp.int32),
        pltpu.VMEM((_NCHUNK, _CHUNK), jnp.int32),
        pltpu.VMEM((_NCHUNK, _CHUNK), jnp.int32),
        pltpu.VMEM((_NCHUNK, _CHUNK, _D), jnp.float32),
        pltpu.VMEM((_NCHUNK, _CHUNK, _D), jnp.float32),
        pltpu.VMEM((_NCHUNK, _CHUNK, _D), jnp.float32),
        pltpu.VMEM((_BPW,), jnp.float32),
        pltpu.SemaphoreType.DMA,
        pltpu.SemaphoreType.DMA,
        pltpu.SemaphoreType.DMA,
        pltpu.SemaphoreType.DMA,
    ],
)
def _bt_norm_kernel(i_hbm, j_hbm, k_hbm, u_hbm, v_hbm, out_hbm,
                    ii_v, jj_v, kk_v, u_v, vj_v, vk_v, o_v,
                    sem0, sem1, sem2, sem3):
    sems = (sem0, sem1, sem2, sem3)
    wid = lax.axis_index("s") * _NC + lax.axis_index("c")
    base = wid * _BPW

    idx_copies = []
    for c in range(_NCHUNK):
        off = base + c * _CHUNK
        idx_copies.append(pltpu.async_copy(i_hbm.at[pl.ds(off, _CHUNK)], ii_v.at[c], sem0))
        idx_copies.append(pltpu.async_copy(j_hbm.at[pl.ds(off, _CHUNK)], jj_v.at[c], sem0))
        idx_copies.append(pltpu.async_copy(k_hbm.at[pl.ds(off, _CHUNK)], kk_v.at[c], sem0))
    for cp in idx_copies:
        cp.wait()

    copies = []
    for c in range(_NCHUNK):
        copies.append((
            pltpu.async_copy(u_hbm.at[ii_v.at[c]], u_v.at[c], sems[c]),
            pltpu.async_copy(v_hbm.at[jj_v.at[c]], vj_v.at[c], sems[c]),
            pltpu.async_copy(v_hbm.at[kk_v.at[c]], vk_v.at[c], sems[c]),
        ))

    lane = lax.iota(jnp.int32, _L)
    hi_mask = lane == (_L - 1)

    for c in range(_NCHUNK):
        for cp in copies[c]:
            cp.wait()

        def row_body(r, _, c=c):
            acc = jnp.zeros((_L,), jnp.float32)
            for q in range(_D // _L):
                sl = pl.ds(q * _L, _L)
                u16 = u_v[c, r, sl]
                dj = u16 - vj_v[c, r, sl]
                dk = u16 - vk_v[c, r, sl]
                acc = acc + (dk * dk - dj * dj)
            cum = plsc.cumsum(acc)
            sig = 1.0 / (1.0 + jnp.exp(-cum))
            pos = jnp.full((_L,), c * _CHUNK + r, jnp.int32)
            plsc.store_scatter(o_v, [pos], sig, mask=hi_mask)
            return 0

        lax.fori_loop(0, _CHUNK, row_body, 0, unroll=4)

    pltpu.sync_copy(o_v, out_hbm.at[pl.ds(base, _BPW)])


def kernel(i, j, k, u_weight, v_weight):
    return _bt_norm_kernel(
        i.astype(jnp.int32),
        j.astype(jnp.int32),
        k.astype(jnp.int32),
        u_weight,
        v_weight,
    )


# speedup vs baseline: 1.1892x; 1.0083x over previous
import functools

import jax
import jax.numpy as jnp
from jax import lax
from jax.experimental import pallas as pl
from jax.experimental.pallas import tpu as pltpu
from jax.experimental.pallas import tpu_sc as plsc

_D = 64
_B = 16384
_L = 16
_NC = 2
_NS = 16
_NW = _NC * _NS
_BPW = _B // _NW
_CHUNK = 512
_NCHUNK = _BPW // _CHUNK

_mesh = plsc.VectorSubcoreMesh(core_axis_name="c", subcore_axis_name="s")


@functools.partial(
    pl.kernel,
    mesh=_mesh,
    out_type=jax.ShapeDtypeStruct((_B,), jnp.float32),
    compiler_params=pltpu.CompilerParams(
        needs_layout_passes=False, use_tc_tiling_on_sc=False
    ),
    scratch_types=[
        pltpu.VMEM((_NCHUNK, _CHUNK), jnp.int32),
        pltpu.VMEM((_NCHUNK, _CHUNK), jnp.int32),
        pltpu.VMEM((_NCHUNK, _CHUNK), jnp.int32),
        pltpu.VMEM((_NCHUNK, _CHUNK, _D), jnp.float32),
        pltpu.VMEM((_NCHUNK, _CHUNK, _D), jnp.float32),
        pltpu.VMEM((_NCHUNK, _CHUNK, _D), jnp.float32),
        pltpu.VMEM((_BPW,), jnp.float32),
        pltpu.SemaphoreType.DMA,
        pltpu.SemaphoreType.DMA,
        pltpu.SemaphoreType.DMA,
        pltpu.SemaphoreType.DMA,
    ],
)
def _bt_norm_kernel(i_hbm, j_hbm, k_hbm, u_hbm, v_hbm, out_hbm,
                    ii_v, jj_v, kk_v, u_v, vj_v, vk_v, o_v,
                    sem0, sem1, sem2, sem3):
    sems = (sem0, sem1, sem2, sem3)
    wid = lax.axis_index("s") * _NC + lax.axis_index("c")
    base = wid * _BPW

    idx_copies = []
    for c in range(_NCHUNK):
        off = base + c * _CHUNK
        idx_copies.append(pltpu.async_copy(i_hbm.at[pl.ds(off, _CHUNK)], ii_v.at[c], sem0))
        idx_copies.append(pltpu.async_copy(j_hbm.at[pl.ds(off, _CHUNK)], jj_v.at[c], sem0))
        idx_copies.append(pltpu.async_copy(k_hbm.at[pl.ds(off, _CHUNK)], kk_v.at[c], sem0))
    for cp in idx_copies:
        cp.wait()

    copies = []
    for c in range(_NCHUNK):
        copies.append((
            pltpu.async_copy(u_hbm.at[ii_v.at[c]], u_v.at[c], sems[c]),
            pltpu.async_copy(v_hbm.at[jj_v.at[c]], vj_v.at[c], sems[c]),
            pltpu.async_copy(v_hbm.at[kk_v.at[c]], vk_v.at[c], sems[c]),
        ))

    lane = lax.iota(jnp.int32, _L)
    hi_mask = lane == (_L - 1)

    for c in range(_NCHUNK):
        for cp in copies[c]:
            cp.wait()

        def row_body(r, _, c=c):
            acc = jnp.zeros((_L,), jnp.float32)
            for q in range(_D // _L):
                sl = pl.ds(q * _L, _L)
                u16 = u_v[c, r, sl]
                dj = u16 - vj_v[c, r, sl]
                dk = u16 - vk_v[c, r, sl]
                acc = acc + (dk * dk - dj * dj)
            cum = plsc.cumsum(acc)
            sig = 1.0 / (1.0 + jnp.exp(-cum))
            pos = jnp.full((_L,), c * _CHUNK + r, jnp.int32)
            plsc.store_scatter(o_v, [pos], sig, mask=hi_mask)
            return 0

        lax.fori_loop(0, _CHUNK, row_body, 0, unroll=4)

    pltpu.sync_copy(o_v, out_hbm.at[pl.ds(base, _BPW)])


def kernel(i, j, k, u_weight, v_weight):
    return _bt_norm_kernel(
        i.astype(jnp.int32),
        j.astype(jnp.int32),
        k.astype(jnp.int32),
        u_weight,
        v_weight,
    )


# speedup vs baseline: 1.5499x; 1.3033x over previous
import functools

import jax
import jax.numpy as jnp
from jax import lax
from jax.experimental import pallas as pl
from jax.experimental.pallas import tpu as pltpu
from jax.experimental.pallas import tpu_sc as plsc

_D = 64
_B = 16384
_L = 16
_NC = 2
_NS = 16
_NW = _NC * _NS
_BPW = _B // _NW
_CHUNK = 256
_NCHUNK = _BPW // _CHUNK

_mesh = plsc.VectorSubcoreMesh(core_axis_name="c", subcore_axis_name="s")


@functools.partial(
    pl.kernel,
    mesh=_mesh,
    out_type=jax.ShapeDtypeStruct((_B,), jnp.float32),
    compiler_params=pltpu.CompilerParams(needs_layout_passes=False),
    scratch_types=[
        pltpu.VMEM((_BPW,), jnp.int32),
        pltpu.VMEM((_BPW,), jnp.int32),
        pltpu.VMEM((_BPW,), jnp.int32),
        pltpu.VMEM((_CHUNK, _D), jnp.float32),
        pltpu.VMEM((_CHUNK, _D), jnp.float32),
        pltpu.VMEM((_CHUNK, _D), jnp.float32),
        pltpu.VMEM((_BPW,), jnp.float32),
        pltpu.SemaphoreType.DMA,
        pltpu.SemaphoreType.DMA,
        pltpu.SemaphoreType.DMA,
        pltpu.SemaphoreType.DMA,
    ],
)
def _bt_norm_kernel(i_hbm, j_hbm, k_hbm, u_hbm, v_hbm, out_hbm,
                    ii_v, jj_v, kk_v, u_b, vj_b, vk_b, o_v,
                    s_idx, s_u, s_vj, s_vk):
    wid = lax.axis_index("s") * _NC + lax.axis_index("c")
    base = wid * _BPW

    c1 = pltpu.async_copy(i_hbm.at[pl.ds(base, _BPW)], ii_v, s_idx)
    c2 = pltpu.async_copy(j_hbm.at[pl.ds(base, _BPW)], jj_v, s_idx)
    c3 = pltpu.async_copy(k_hbm.at[pl.ds(base, _BPW)], kk_v, s_idx)
    c1.wait()
    c2.wait()
    c3.wait()

    lane = lax.iota(jnp.int32, _L)
    hi_mask = lane == (_L - 1)

    for c in range(_NCHUNK):
        def enq_body(g, _, c=c):
            row0 = c * _CHUNK + g * _L
            iv = ii_v[pl.ds(row0, _L)]
            jv = jj_v[pl.ds(row0, _L)]
            kv = kk_v[pl.ds(row0, _L)]
            for t in range(_L):
                r = g * _L + t
                pltpu.async_copy(
                    u_hbm.at[pl.ds(iv[t], 1)], u_b.at[pl.ds(r, 1)], s_u)
                pltpu.async_copy(
                    v_hbm.at[pl.ds(jv[t], 1)], vj_b.at[pl.ds(r, 1)], s_vj)
                pltpu.async_copy(
                    v_hbm.at[pl.ds(kv[t], 1)], vk_b.at[pl.ds(r, 1)], s_vk)
            return 0

        lax.fori_loop(0, _CHUNK // _L, enq_body, 0)

        pltpu.make_async_copy(u_hbm.at[pl.ds(0, _CHUNK)], u_b, s_u).wait()
        pltpu.make_async_copy(v_hbm.at[pl.ds(0, _CHUNK)], vj_b, s_vj).wait()
        pltpu.make_async_copy(v_hbm.at[pl.ds(0, _CHUNK)], vk_b, s_vk).wait()

        def row_body(r, _, c=c):
            acc = jnp.zeros((_L,), jnp.float32)
            for q in range(_D // _L):
                sl = pl.ds(q * _L, _L)
                u16 = u_b[r, sl]
                dj = u16 - vj_b[r, sl]
                dk = u16 - vk_b[r, sl]
                acc = acc + (dk * dk - dj * dj)
            cum = plsc.cumsum(acc)
            sig = 1.0 / (1.0 + jnp.exp(-cum))
            pos = jnp.full((_L,), c * _CHUNK + r, jnp.int32)
            plsc.store_scatter(o_v, [pos], sig, mask=hi_mask)
            return 0

        lax.fori_loop(0, _CHUNK, row_body, 0, unroll=4)

    pltpu.sync_copy(o_v, out_hbm.at[pl.ds(base, _BPW)])


def kernel(i, j, k, u_weight, v_weight):
    return _bt_norm_kernel(
        i.astype(jnp.int32),
        j.astype(jnp.int32),
        k.astype(jnp.int32),
        u_weight,
        v_weight,
    )


# speedup vs baseline: 1.6074x; 1.0371x over previous
import functools

import jax
import jax.numpy as jnp
from jax import lax
from jax.experimental import pallas as pl
from jax.experimental.pallas import tpu as pltpu
from jax.experimental.pallas import tpu_sc as plsc

_D = 64
_B = 16384
_L = 16
_NC = 2
_NS = 16
_NW = _NC * _NS
_BPW = _B // _NW
_CHUNK = 128
_NCHUNK = _BPW // _CHUNK

_mesh = plsc.VectorSubcoreMesh(core_axis_name="c", subcore_axis_name="s")


@functools.partial(
    pl.kernel,
    mesh=_mesh,
    out_type=jax.ShapeDtypeStruct((_B,), jnp.float32),
    compiler_params=pltpu.CompilerParams(needs_layout_passes=False),
    scratch_types=[
        pltpu.VMEM((_BPW,), jnp.int32),
        pltpu.VMEM((_BPW,), jnp.int32),
        pltpu.VMEM((_BPW,), jnp.int32),
        pltpu.VMEM((2, _CHUNK, _D), jnp.float32),
        pltpu.VMEM((2, _CHUNK, _D), jnp.float32),
        pltpu.VMEM((2, _CHUNK, _D), jnp.float32),
        pltpu.VMEM((_BPW,), jnp.float32),
        pltpu.SemaphoreType.DMA,
        pltpu.SemaphoreType.DMA,
        pltpu.SemaphoreType.DMA,
        pltpu.SemaphoreType.DMA,
        pltpu.SemaphoreType.DMA,
        pltpu.SemaphoreType.DMA,
        pltpu.SemaphoreType.DMA,
    ],
)
def _bt_norm_kernel(i_hbm, j_hbm, k_hbm, u_hbm, v_hbm, out_hbm,
                    ii_v, jj_v, kk_v, u_b, vj_b, vk_b, o_v,
                    s_idx, s_u0, s_vj0, s_vk0, s_u1, s_vj1, s_vk1):
    sem_sets = ((s_u0, s_vj0, s_vk0), (s_u1, s_vj1, s_vk1))
    wid = lax.axis_index("s") * _NC + lax.axis_index("c")
    base = wid * _BPW

    c1 = pltpu.async_copy(i_hbm.at[pl.ds(base, _BPW)], ii_v, s_idx)
    c2 = pltpu.async_copy(j_hbm.at[pl.ds(base, _BPW)], jj_v, s_idx)
    c3 = pltpu.async_copy(k_hbm.at[pl.ds(base, _BPW)], kk_v, s_idx)
    c1.wait()
    c2.wait()
    c3.wait()

    lane = lax.iota(jnp.int32, _L)
    hi_mask = lane == (_L - 1)

    def enqueue_chunk(c, buf):
        s_u, s_vj, s_vk = sem_sets[buf]

        def enq_body(g, _):
            row0 = c * _CHUNK + g * _L
            iv = ii_v[pl.ds(row0, _L)]
            jv = jj_v[pl.ds(row0, _L)]
            kv = kk_v[pl.ds(row0, _L)]
            for t in range(_L):
                r = g * _L + t
                pltpu.async_copy(
                    u_hbm.at[pl.ds(iv[t], 1)], u_b.at[buf, pl.ds(r, 1)], s_u)
                pltpu.async_copy(
                    v_hbm.at[pl.ds(jv[t], 1)], vj_b.at[buf, pl.ds(r, 1)], s_vj)
                pltpu.async_copy(
                    v_hbm.at[pl.ds(kv[t], 1)], vk_b.at[buf, pl.ds(r, 1)], s_vk)
            return 0

        lax.fori_loop(0, _CHUNK // _L, enq_body, 0)

    def drain_chunk(buf):
        s_u, s_vj, s_vk = sem_sets[buf]
        pltpu.make_async_copy(
            u_hbm.at[pl.ds(0, _CHUNK)], u_b.at[buf], s_u).wait()
        pltpu.make_async_copy(
            v_hbm.at[pl.ds(0, _CHUNK)], vj_b.at[buf], s_vj).wait()
        pltpu.make_async_copy(
            v_hbm.at[pl.ds(0, _CHUNK)], vk_b.at[buf], s_vk).wait()

    def compute_chunk(c, buf):
        def row_body(r, _):
            acc = jnp.zeros((_L,), jnp.float32)
            for q in range(_D // _L):
                sl = pl.ds(q * _L, _L)
                u16 = u_b[buf, r, sl]
                dj = u16 - vj_b[buf, r, sl]
                dk = u16 - vk_b[buf, r, sl]
                acc = acc + (dk * dk - dj * dj)
            cum = plsc.cumsum(acc)
            sig = 1.0 / (1.0 + jnp.exp(-cum))
            pos = jnp.full((_L,), c * _CHUNK + r, jnp.int32)
            plsc.store_scatter(o_v, [pos], sig, mask=hi_mask)
            return 0

        lax.fori_loop(0, _CHUNK, row_body, 0, unroll=4)

    enqueue_chunk(0, 0)
    for c in range(_NCHUNK):
        buf = c % 2
        drain_chunk(buf)
        if c + 1 < _NCHUNK:
            enqueue_chunk(c + 1, (c + 1) % 2)
        compute_chunk(c, buf)

    pltpu.sync_copy(o_v, out_hbm.at[pl.ds(base, _BPW)])


def kernel(i, j, k, u_weight, v_weight):
    return _bt_norm_kernel(
        i.astype(jnp.int32),
        j.astype(jnp.int32),
        k.astype(jnp.int32),
        u_weight,
        v_weight,
    )
